# baseline (speedup 1.0000x reference)
import functools

import jax
import jax.numpy as jnp
from jax import lax
from jax.experimental import pallas as pl
from jax.experimental.pallas import tpu as pltpu, tpu_sc as plsc

N = 10000
E = 320000
D = 128
R = 64

NPAD = 10112
NC = 2
NS = 16
NW = NC * NS
EW = E // NW
BLK = 80
NBLK = EW // BLK
SEGB = 5
HA = 48
HB = 32
SEG = NPAD // NS
TCB = NPAD


def _tc_pre_body(x_ref, wn_ref, wr_ref, rt_ref, bn_ref, br_ref, a3_ref,
                 mnode_ref, ssrc_ref, mrel_ref, srel_ref):
    xb = x_ref[...]
    mnode_ref[...] = (
        jnp.dot(xb, wn_ref[...], preferred_element_type=jnp.float32)
        + bn_ref[...]
    )
    a = a3_ref[...]
    ssrc_ref[...] = lax.dot_general(a[0:1, :], xb, (((1,), (1,)), ((), ())),
                                    preferred_element_type=jnp.float32)
    rt = rt_ref[...]
    mrel_ref[...] = (
        jnp.dot(rt, wr_ref[...], preferred_element_type=jnp.float32)
        + br_ref[...]
    )
    srel_ref[...] = lax.dot_general(a[2:3, :], rt, (((1,), (1,)), ((), ())),
                                    preferred_element_type=jnp.float32)


def _tc_pre(x_pad, W_node_w, W_rel_w, rel_table, b_node, b_rel, attn3):
    return pl.pallas_call(
        _tc_pre_body,
        out_shape=[
            jax.ShapeDtypeStruct((NPAD, D), jnp.float32),
            jax.ShapeDtypeStruct((1, NPAD), jnp.float32),
            jax.ShapeDtypeStruct((R, D), jnp.float32),
            jax.ShapeDtypeStruct((1, R), jnp.float32),
        ],
    )(x_pad, W_node_w, W_rel_w, rel_table, b_node, b_rel, attn3)


def _sc_body(mnode_hbm, ssrc_hbm, srel_hbm, mrel_hbm, src_hbm, dst_hbm,
             rel_hbm, acc_hbm, den_hbm,
             ssrc_v, srel_v, sidx_v, didx_v, ridx_v, dscatA_v, dscatB_v, w_v,
             mnA_v, mrA_v, mnB_v, mrB_v, den_v, acc_sp,
             semA, semB, semSA, semSB):
    cid = lax.axis_index("c")
    sid = lax.axis_index("s")
    wid = cid * NS + sid

    pltpu.sync_copy(ssrc_hbm, ssrc_v)
    pltpu.sync_copy(srel_hbm, srel_v)

    zeros16 = jnp.zeros((16,), jnp.float32)

    def _zero_rows(r, carry):
        for c in range(D // 16):
            mnA_v[r, pl.ds(c * 16, 16)] = zeros16
        return carry

    lax.fori_loop(0, HA, _zero_rows, 0)

    def _zero_rows_b(r, carry):
        for c in range(D // 16):
            mnB_v[r, pl.ds(c * 16, 16)] = zeros16
        return carry

    lax.fori_loop(0, HB, _zero_rows_b, 0)

    for j in range(HA // 16):
        dscatA_v[pl.ds(j * 16, 16)] = jnp.zeros((16,), jnp.int32)
    for j in range(HB // 16):
        dscatB_v[pl.ds(j * 16, 16)] = jnp.zeros((16,), jnp.int32)

    def _zero_den(k, carry):
        den_v[pl.ds(k * 16, 16)] = zeros16
        return carry

    lax.fori_loop(0, NPAD // 16, _zero_den, 0)

    def _zero_seg(k, carry):
        pltpu.sync_copy(mnA_v, acc_sp.at[pl.ds(sid * SEG + k * HA, HA)])
        return carry

    lax.fori_loop(0, SEG // HA, _zero_seg, 0)
    rem = SEG - (SEG // HA) * HA
    if rem:
        pltpu.sync_copy(mnA_v.at[pl.ds(0, rem)],
                        acc_sp.at[pl.ds(sid * SEG + SEG - rem, rem)])
    plsc.subcore_barrier()

    pltpu.async_copy(mnA_v, acc_sp.at[dscatA_v], semSA, add=True)
    pltpu.async_copy(mnB_v, acc_sp.at[dscatB_v], semSB, add=True)

    def _segment(seg, carry0):
        base = pl.multiple_of(wid * EW + seg * (SEGB * BLK), 16)
        pltpu.sync_copy(src_hbm.at[pl.ds(base, SEGB * BLK)], sidx_v)
        pltpu.sync_copy(dst_hbm.at[pl.ds(base, SEGB * BLK)], didx_v)
        pltpu.sync_copy(rel_hbm.at[pl.ds(base, SEGB * BLK)], ridx_v)

        def _block(q, carry):
            off = pl.multiple_of(q * BLK, 16)
            offB = pl.multiple_of(q * BLK + HA, 16)
            pltpu.make_async_copy(mnA_v, acc_sp.at[dscatA_v], semSA).wait()
            pltpu.make_async_copy(mnB_v, acc_sp.at[dscatB_v], semSB).wait()
            pltpu.async_copy(mnode_hbm.at[sidx_v.at[pl.ds(off, HA)]],
                             mnA_v, semA)
            pltpu.async_copy(mrel_hbm.at[ridx_v.at[pl.ds(off, HA)]],
                             mrA_v, semA)
            pltpu.async_copy(mnode_hbm.at[sidx_v.at[pl.ds(offB, HB)]],
                             mnB_v, semB)
            pltpu.async_copy(mrel_hbm.at[ridx_v.at[pl.ds(offB, HB)]],
                             mrB_v, semB)

            def _score(j, c2):
                j16 = pl.multiple_of(j * 16, 16)
                s16 = sidx_v[pl.ds(off + j16, 16)]
                d16 = didx_v[pl.ds(off + j16, 16)]
                r16 = ridx_v[pl.ds(off + j16, 16)]
                w16 = jnp.exp(plsc.load_gather(ssrc_v, [s16])
                              + plsc.load_gather(srel_v, [r16]))
                w_v[pl.ds(j16, 16)] = w16
                plsc.addupdate_scatter(den_v, [d16], w16)
                return c2

            lax.fori_loop(0, BLK // 16, _score, 0)

            def _stage_a(j, c2):
                j16 = pl.multiple_of(j * 16, 16)
                dscatA_v[pl.ds(j16, 16)] = didx_v[pl.ds(off + j16, 16)]
                return c2

            lax.fori_loop(0, HA // 16, _stage_a, 0)

            def _stage_b(j, c2):
                j16 = pl.multiple_of(j * 16, 16)
                dscatB_v[pl.ds(j16, 16)] = didx_v[pl.ds(offB + j16, 16)]
                return c2

            lax.fori_loop(0, HB // 16, _stage_b, 0)

            pltpu.make_async_copy(mnode_hbm.at[sidx_v.at[pl.ds(off, HA)]],
                                  mnA_v, semA).wait()
            pltpu.make_async_copy(mrel_hbm.at[ridx_v.at[pl.ds(off, HA)]],
                                  mrA_v, semA).wait()

            def _groupA(j, c2):
                w16 = w_v[pl.ds(j * 16, 16)]
                for i in range(16):
                    l = j * 16 + i
                    wb = jnp.take_along_axis(
                        w16, jnp.full((16,), i, jnp.int32), axis=0,
                        mode="promise_in_bounds")
                    for c in range(D // 16):
                        rowc = mnA_v[l, pl.ds(c * 16, 16)]
                        mrc = mrA_v[l, pl.ds(c * 16, 16)]
                        mnA_v[l, pl.ds(c * 16, 16)] = (rowc + mrc) * wb
                return c2

            lax.fori_loop(0, HA // 16, _groupA, 0)
            pltpu.async_copy(mnA_v, acc_sp.at[dscatA_v], semSA, add=True)

            pltpu.make_async_copy(mnode_hbm.at[sidx_v.at[pl.ds(offB, HB)]],
                                  mnB_v, semB).wait()
            pltpu.make_async_copy(mrel_hbm.at[ridx_v.at[pl.ds(offB, HB)]],
                                  mrB_v, semB).wait()

            def _groupB(j, c2):
                w16 = w_v[pl.ds(HA + j * 16, 16)]
                for i in range(16):
                    l = j * 16 + i
                    wb = jnp.take_along_axis(
                        w16, jnp.full((16,), i, jnp.int32), axis=0,
                        mode="promise_in_bounds")
                    for c in range(D // 16):
                        rowc = mnB_v[l, pl.ds(c * 16, 16)]
                        mrc = mrB_v[l, pl.ds(c * 16, 16)]
                        mnB_v[l, pl.ds(c * 16, 16)] = (rowc + mrc) * wb
                return c2

            lax.fori_loop(0, HB // 16, _groupB, 0)
            pltpu.async_copy(mnB_v, acc_sp.at[dscatB_v], semSB, add=True)
            return carry

        lax.fori_loop(0, SEGB, _block, 0)
        return carry0

    lax.fori_loop(0, NBLK // SEGB, _segment, 0)

    pltpu.make_async_copy(mnA_v, acc_sp.at[dscatA_v], semSA).wait()
    pltpu.make_async_copy(mnB_v, acc_sp.at[dscatB_v], semSB).wait()
    plsc.subcore_barrier()

    pltpu.sync_copy(acc_sp.at[pl.ds(sid * SEG, SEG)], acc_hbm.at[cid, sid])
    pltpu.sync_copy(den_v, den_hbm.at[cid, sid])


def _sc_edge_pass(m_node, s_src, s_rel, m_rel, src, dst, rel):
    mesh = plsc.VectorSubcoreMesh(core_axis_name="c", subcore_axis_name="s")
    call = functools.partial(
        pl.kernel,
        mesh=mesh,
        compiler_params=pltpu.CompilerParams(needs_layout_passes=False),
        out_type=[
            jax.ShapeDtypeStruct((NC, NS, SEG, D), jnp.float32),
            jax.ShapeDtypeStruct((NC, NS, NPAD), jnp.float32),
        ],
        scratch_types=[
            pltpu.VMEM((NPAD,), jnp.float32),
            pltpu.VMEM((R,), jnp.float32),
            pltpu.VMEM((SEGB * BLK,), jnp.int32),
            pltpu.VMEM((SEGB * BLK,), jnp.int32),
            pltpu.VMEM((SEGB * BLK,), jnp.int32),
            pltpu.VMEM((HA,), jnp.int32),
            pltpu.VMEM((HB,), jnp.int32),
            pltpu.VMEM((BLK,), jnp.float32),
            pltpu.VMEM((HA, D), jnp.float32),
            pltpu.VMEM((HA, D), jnp.float32),
            pltpu.VMEM((HB, D), jnp.float32),
            pltpu.VMEM((HB, D), jnp.float32),
            pltpu.VMEM((NPAD,), jnp.float32),
            pltpu.VMEM_SHARED((NPAD, D), jnp.float32),
            pltpu.SemaphoreType.DMA,
            pltpu.SemaphoreType.DMA,
            pltpu.SemaphoreType.DMA,
            pltpu.SemaphoreType.DMA,
        ],
    )(_sc_body)
    return call(m_node, s_src, s_rel, m_rel, src, dst, rel)


def _tc_combine_body(acc_ref, den_ref, x_ref, out_ref):
    a = acc_ref[...]
    msg = a[0] + a[1]
    den = jnp.sum(den_ref[...], axis=0)[:, None]
    out_ref[...] = jnp.where(den > 0.0, msg / den, x_ref[...])


def _tc_combine(acc, den, x_pad):
    return pl.pallas_call(
        _tc_combine_body,
        out_shape=jax.ShapeDtypeStruct((NPAD, D), jnp.float32),
    )(acc, den, x_pad)


def kernel(x, rel_table, edge_index, edge_rel, W_node_w, W_node_b, W_rel_w,
           W_rel_b, attn_w, attn_b):
    x_pad = jnp.pad(x, ((0, NPAD - N), (0, 0)))
    attn3 = attn_w.reshape(3, D)
    b_node = W_node_b.reshape(1, D)
    b_rel = W_rel_b.reshape(1, D)

    m_node, s_src, m_rel, s_rel = _tc_pre(
        x_pad, W_node_w, W_rel_w, rel_table, b_node, b_rel, attn3)

    acc, den = _sc_edge_pass(m_node, s_src.reshape(NPAD), s_rel.reshape(R),
                             m_rel, edge_index[0], edge_index[1], edge_rel)

    out_pad = _tc_combine(acc.reshape(NC, NPAD, D), den.reshape(NW, NPAD),
                          x_pad)
    return out_pad[:N]


# speedup vs baseline: 1.0620x; 1.0620x over previous
import functools

import jax
import jax.numpy as jnp
from jax import lax
from jax.experimental import pallas as pl
from jax.experimental.pallas import tpu as pltpu, tpu_sc as plsc

N = 10000
E = 320000
D = 128
R = 64

NPAD = 10112
NC = 2
NS = 16
NW = NC * NS
EW = E // NW
BLK = 80
NBLK = EW // BLK
SEGB = 25
SEG = NPAD // NS
TCB = NPAD


def _tc_pre_body(x_ref, wn_ref, wr_ref, rt_ref, bn_ref, br_ref, a3_ref,
                 mnode_ref, ssrc_ref, mrel_ref, srel_ref):
    xb = x_ref[...]
    mnode_ref[...] = (
        jnp.dot(xb, wn_ref[...], preferred_element_type=jnp.float32)
        + bn_ref[...]
    )
    a = a3_ref[...]
    ssrc_ref[...] = lax.dot_general(a[0:1, :], xb, (((1,), (1,)), ((), ())),
                                    preferred_element_type=jnp.float32)
    rt = rt_ref[...]
    mrel_ref[...] = (
        jnp.dot(rt, wr_ref[...], preferred_element_type=jnp.float32)
        + br_ref[...]
    )
    srel_ref[...] = lax.dot_general(a[2:3, :], rt, (((1,), (1,)), ((), ())),
                                    preferred_element_type=jnp.float32)


def _tc_pre(x_pad, W_node_w, W_rel_w, rel_table, b_node, b_rel, attn3):
    return pl.pallas_call(
        _tc_pre_body,
        out_shape=[
            jax.ShapeDtypeStruct((NPAD, D), jnp.float32),
            jax.ShapeDtypeStruct((1, NPAD), jnp.float32),
            jax.ShapeDtypeStruct((R, D), jnp.float32),
            jax.ShapeDtypeStruct((1, R), jnp.float32),
        ],
    )(x_pad, W_node_w, W_rel_w, rel_table, b_node, b_rel, attn3)


def _sc_body(mnode_hbm, ssrc_hbm, srel_hbm, mrel_hbm, src_hbm, dst_hbm,
             rel_hbm, acc_hbm, den_hbm,
             ssrc_v, srel_v, sidx_v, didx_v, ridx_v, dscat_v, w_v,
             rowsg_v, rows2_v, den_v, acc_sp, semn, semr):
    cid = lax.axis_index("c")
    sid = lax.axis_index("s")
    wid = cid * NS + sid

    pltpu.sync_copy(ssrc_hbm, ssrc_v)
    pltpu.sync_copy(srel_hbm, srel_v)

    zeros16 = jnp.zeros((16,), jnp.float32)

    def _zero_rows(r, carry):
        for c in range(D // 16):
            rowsg_v[r, pl.ds(c * 16, 16)] = zeros16
        return carry

    lax.fori_loop(0, BLK, _zero_rows, 0)

    def _zero_den(k, carry):
        den_v[pl.ds(k * 16, 16)] = zeros16
        return carry

    lax.fori_loop(0, NPAD // 16, _zero_den, 0)

    def _zero_seg(k, carry):
        pltpu.sync_copy(rowsg_v, acc_sp.at[pl.ds(sid * SEG + k * BLK, BLK)])
        return carry

    lax.fori_loop(0, SEG // BLK, _zero_seg, 0)
    rem = SEG - (SEG // BLK) * BLK
    if rem:
        pltpu.sync_copy(rowsg_v.at[pl.ds(0, rem)],
                        acc_sp.at[pl.ds(sid * SEG + SEG - rem, rem)])
    plsc.subcore_barrier()

    def _segment(seg, carry0):
        base = pl.multiple_of(wid * EW + seg * (SEGB * BLK), 16)
        pltpu.sync_copy(src_hbm.at[pl.ds(base, SEGB * BLK)], sidx_v)
        pltpu.sync_copy(dst_hbm.at[pl.ds(base, SEGB * BLK)], didx_v)
        pltpu.sync_copy(rel_hbm.at[pl.ds(base, SEGB * BLK)], ridx_v)

        def _block(q, carry):
            off = pl.multiple_of(q * BLK, 16)
            cpn = pltpu.async_copy(mnode_hbm.at[sidx_v.at[pl.ds(off, BLK)]],
                                   rowsg_v, semn)
            cpr = pltpu.async_copy(mrel_hbm.at[ridx_v.at[pl.ds(off, BLK)]],
                                   rows2_v, semr)

            for j in range(BLK // 16):
                s16 = sidx_v[pl.ds(off + j * 16, 16)]
                d16 = didx_v[pl.ds(off + j * 16, 16)]
                r16 = ridx_v[pl.ds(off + j * 16, 16)]
                w16 = jnp.exp(plsc.load_gather(ssrc_v, [s16])
                              + plsc.load_gather(srel_v, [r16]))
                w_v[pl.ds(j * 16, 16)] = w16
                dscat_v[pl.ds(j * 16, 16)] = d16
                plsc.addupdate_scatter(den_v, [d16], w16)

            cpn.wait()
            cpr.wait()

            def _group(j, c2):
                w16 = w_v[pl.ds(j * 16, 16)]
                for i in range(16):
                    l = j * 16 + i
                    wb = jnp.take_along_axis(
                        w16, jnp.full((16,), i, jnp.int32), axis=0,
                        mode="promise_in_bounds")
                    for c in range(D // 16):
                        rowc = rowsg_v[l, pl.ds(c * 16, 16)]
                        mrc = rows2_v[l, pl.ds(c * 16, 16)]
                        rowsg_v[l, pl.ds(c * 16, 16)] = (rowc + mrc) * wb
                return c2

            lax.fori_loop(0, BLK // 16, _group, 0)

            pltpu.sync_copy(rowsg_v, acc_sp.at[dscat_v], add=True)
            return carry

        lax.fori_loop(0, SEGB, _block, 0)
        return carry0

    lax.fori_loop(0, NBLK // SEGB, _segment, 0)
    plsc.subcore_barrier()

    pltpu.sync_copy(acc_sp.at[pl.ds(sid * SEG, SEG)], acc_hbm.at[cid, sid])
    pltpu.sync_copy(den_v, den_hbm.at[cid, sid])


def _sc_edge_pass(m_node, s_src, s_rel, m_rel, src, dst, rel):
    mesh = plsc.VectorSubcoreMesh(core_axis_name="c", subcore_axis_name="s")
    call = functools.partial(
        pl.kernel,
        mesh=mesh,
        compiler_params=pltpu.CompilerParams(needs_layout_passes=False),
        out_type=[
            jax.ShapeDtypeStruct((NC, NS, SEG, D), jnp.float32),
            jax.ShapeDtypeStruct((NC, NS, NPAD), jnp.float32),
        ],
        scratch_types=[
            pltpu.VMEM((NPAD,), jnp.float32),
            pltpu.VMEM((R,), jnp.float32),
            pltpu.VMEM((SEGB * BLK,), jnp.int32),
            pltpu.VMEM((SEGB * BLK,), jnp.int32),
            pltpu.VMEM((SEGB * BLK,), jnp.int32),
            pltpu.VMEM((BLK,), jnp.int32),
            pltpu.VMEM((BLK,), jnp.float32),
            pltpu.VMEM((BLK, D), jnp.float32),
            pltpu.VMEM((BLK, D), jnp.float32),
            pltpu.VMEM((NPAD,), jnp.float32),
            pltpu.VMEM_SHARED((NPAD, D), jnp.float32),
            pltpu.SemaphoreType.DMA,
            pltpu.SemaphoreType.DMA,
        ],
    )(_sc_body)
    return call(m_node, s_src, s_rel, m_rel, src, dst, rel)


def _tc_combine_body(acc_ref, den_ref, x_ref, out_ref):
    a = acc_ref[...]
    msg = a[0] + a[1]
    den = jnp.sum(den_ref[...], axis=0)[:, None]
    out_ref[...] = jnp.where(den > 0.0, msg / den, x_ref[...])


def _tc_combine(acc, den, x_pad):
    return pl.pallas_call(
        _tc_combine_body,
        out_shape=jax.ShapeDtypeStruct((NPAD, D), jnp.float32),
    )(acc, den, x_pad)


def kernel(x, rel_table, edge_index, edge_rel, W_node_w, W_node_b, W_rel_w,
           W_rel_b, attn_w, attn_b):
    x_pad = jnp.pad(x, ((0, NPAD - N), (0, 0)))
    attn3 = attn_w.reshape(3, D)
    b_node = W_node_b.reshape(1, D)
    b_rel = W_rel_b.reshape(1, D)

    m_node, s_src, m_rel, s_rel = _tc_pre(
        x_pad, W_node_w, W_rel_w, rel_table, b_node, b_rel, attn3)

    acc, den = _sc_edge_pass(m_node, s_src.reshape(NPAD), s_rel.reshape(R),
                             m_rel, edge_index[0], edge_index[1], edge_rel)

    out_pad = _tc_combine(acc.reshape(NC, NPAD, D), den.reshape(NW, NPAD),
                          x_pad)
    return out_pad[:N]


# speedup vs baseline: 1.1653x; 1.0973x over previous
import functools

import jax
import jax.numpy as jnp
from jax import lax
from jax.experimental import pallas as pl
from jax.experimental.pallas import tpu as pltpu, tpu_sc as plsc

N = 10000
E = 320000
D = 128
R = 64

NPAD = 10112
NC = 2
NS = 16
NW = NC * NS
EW = E // NW
BLK = 80
NBLK = EW // BLK
SEGB = 25
SEG = NPAD // NS
TCB = NPAD


def _tc_pre_body(x_ref, wn_ref, wr_ref, rt_ref, bn_ref, br_ref, a3_ref,
                 mnode_ref, ssrc_ref, mrel_ref, srel_ref):
    xb = x_ref[...]
    mnode_ref[...] = (
        jnp.dot(xb, wn_ref[...], preferred_element_type=jnp.float32)
        + bn_ref[...]
    )
    a = a3_ref[...]
    ssrc_ref[...] = lax.dot_general(a[0:1, :], xb, (((1,), (1,)), ((), ())),
                                    preferred_element_type=jnp.float32)
    rt = rt_ref[...]
    mrel_ref[...] = (
        jnp.dot(rt, wr_ref[...], preferred_element_type=jnp.float32)
        + br_ref[...]
    )
    srel_ref[...] = lax.dot_general(a[2:3, :], rt, (((1,), (1,)), ((), ())),
                                    preferred_element_type=jnp.float32)


def _tc_pre(x_pad, W_node_w, W_rel_w, rel_table, b_node, b_rel, attn3):
    return pl.pallas_call(
        _tc_pre_body,
        out_shape=[
            jax.ShapeDtypeStruct((NPAD, D), jnp.float32),
            jax.ShapeDtypeStruct((1, NPAD), jnp.float32),
            jax.ShapeDtypeStruct((R, D), jnp.float32),
            jax.ShapeDtypeStruct((1, R), jnp.float32),
        ],
    )(x_pad, W_node_w, W_rel_w, rel_table, b_node, b_rel, attn3)


def _sc_body(mnode_hbm, ssrc_hbm, srel_hbm, mrel_hbm, src_hbm, dst_hbm,
             rel_hbm, acc_hbm, den_hbm,
             ssrc_v, srel_v, sidx_v, didx_v, ridx_v, dscat_v, w_v,
             rowsg_v, rows2_v, den_v, acc_sp, semn, semr):
    cid = lax.axis_index("c")
    sid = lax.axis_index("s")
    wid = cid * NS + sid

    pltpu.sync_copy(ssrc_hbm, ssrc_v)
    pltpu.sync_copy(srel_hbm, srel_v)

    zeros16 = jnp.zeros((16,), jnp.float32)

    def _zero_rows(r, carry):
        for c in range(D // 16):
            rowsg_v[r, pl.ds(c * 16, 16)] = zeros16
        return carry

    lax.fori_loop(0, BLK, _zero_rows, 0)

    def _zero_den(k, carry):
        den_v[pl.ds(k * 16, 16)] = zeros16
        return carry

    lax.fori_loop(0, NPAD // 16, _zero_den, 0)

    def _zero_seg(k, carry):
        pltpu.sync_copy(rowsg_v, acc_sp.at[pl.ds(sid * SEG + k * BLK, BLK)])
        return carry

    lax.fori_loop(0, SEG // BLK, _zero_seg, 0)
    rem = SEG - (SEG // BLK) * BLK
    if rem:
        pltpu.sync_copy(rowsg_v.at[pl.ds(0, rem)],
                        acc_sp.at[pl.ds(sid * SEG + SEG - rem, rem)])
    plsc.subcore_barrier()

    def _segment(seg, carry0):
        base = pl.multiple_of(wid * EW + seg * (SEGB * BLK), 16)
        pltpu.sync_copy(src_hbm.at[pl.ds(base, SEGB * BLK)], sidx_v)
        pltpu.sync_copy(dst_hbm.at[pl.ds(base, SEGB * BLK)], didx_v)
        pltpu.sync_copy(rel_hbm.at[pl.ds(base, SEGB * BLK)], ridx_v)

        def _block(q, carry):
            off = pl.multiple_of(q * BLK, 16)
            cpn = pltpu.async_copy(mnode_hbm.at[sidx_v.at[pl.ds(off, BLK)]],
                                   rowsg_v, semn)
            cpr = pltpu.async_copy(mrel_hbm.at[ridx_v.at[pl.ds(off, BLK)]],
                                   rows2_v, semr)

            for j in range(BLK // 16):
                s16 = sidx_v[pl.ds(off + j * 16, 16)]
                d16 = didx_v[pl.ds(off + j * 16, 16)]
                r16 = ridx_v[pl.ds(off + j * 16, 16)]
                w16 = jnp.exp(plsc.load_gather(ssrc_v, [s16])
                              + plsc.load_gather(srel_v, [r16]))
                w_v[pl.ds(j * 16, 16)] = w16
                dscat_v[pl.ds(j * 16, 16)] = d16
                plsc.addupdate_scatter(den_v, [d16], w16)

            cpn.wait()
            cpr.wait()

            @plsc.parallel_loop(0, BLK, step=1, unroll=4)
            def _edge(l):
                g = pl.multiple_of((l >> 4) * 16, 16)
                w16 = w_v[pl.ds(g, 16)]
                wb = jnp.take_along_axis(
                    w16, lax.broadcast_in_dim(l & 15, (16,), ()), axis=0,
                    mode="promise_in_bounds")
                for c in range(D // 16):
                    rowc = rowsg_v[l, pl.ds(c * 16, 16)]
                    mrc = rows2_v[l, pl.ds(c * 16, 16)]
                    rowsg_v[l, pl.ds(c * 16, 16)] = (rowc + mrc) * wb

            pltpu.sync_copy(rowsg_v, acc_sp.at[dscat_v], add=True)
            return carry

        lax.fori_loop(0, SEGB, _block, 0)
        return carry0

    lax.fori_loop(0, NBLK // SEGB, _segment, 0)
    plsc.subcore_barrier()

    pltpu.sync_copy(acc_sp.at[pl.ds(sid * SEG, SEG)], acc_hbm.at[cid, sid])
    pltpu.sync_copy(den_v, den_hbm.at[cid, sid])


def _sc_edge_pass(m_node, s_src, s_rel, m_rel, src, dst, rel):
    mesh = plsc.VectorSubcoreMesh(core_axis_name="c", subcore_axis_name="s")
    call = functools.partial(
        pl.kernel,
        mesh=mesh,
        compiler_params=pltpu.CompilerParams(needs_layout_passes=False),
        out_type=[
            jax.ShapeDtypeStruct((NC, NS, SEG, D), jnp.float32),
            jax.ShapeDtypeStruct((NC, NS, NPAD), jnp.float32),
        ],
        scratch_types=[
            pltpu.VMEM((NPAD,), jnp.float32),
            pltpu.VMEM((R,), jnp.float32),
            pltpu.VMEM((SEGB * BLK,), jnp.int32),
            pltpu.VMEM((SEGB * BLK,), jnp.int32),
            pltpu.VMEM((SEGB * BLK,), jnp.int32),
            pltpu.VMEM((BLK,), jnp.int32),
            pltpu.VMEM((BLK,), jnp.float32),
            pltpu.VMEM((BLK, D), jnp.float32),
            pltpu.VMEM((BLK, D), jnp.float32),
            pltpu.VMEM((NPAD,), jnp.float32),
            pltpu.VMEM_SHARED((NPAD, D), jnp.float32),
            pltpu.SemaphoreType.DMA,
            pltpu.SemaphoreType.DMA,
        ],
    )(_sc_body)
    return call(m_node, s_src, s_rel, m_rel, src, dst, rel)


def _tc_combine_body(acc_ref, den_ref, x_ref, out_ref):
    a = acc_ref[...]
    msg = a[0] + a[1]
    den = jnp.sum(den_ref[...], axis=0)[:, None]
    out_ref[...] = jnp.where(den > 0.0, msg / den, x_ref[...])


def _tc_combine(acc, den, x_pad):
    return pl.pallas_call(
        _tc_combine_body,
        out_shape=jax.ShapeDtypeStruct((NPAD, D), jnp.float32),
    )(acc, den, x_pad)


def kernel(x, rel_table, edge_index, edge_rel, W_node_w, W_node_b, W_rel_w,
           W_rel_b, attn_w, attn_b):
    x_pad = jnp.pad(x, ((0, NPAD - N), (0, 0)))
    attn3 = attn_w.reshape(3, D)
    b_node = W_node_b.reshape(1, D)
    b_rel = W_rel_b.reshape(1, D)

    m_node, s_src, m_rel, s_rel = _tc_pre(
        x_pad, W_node_w, W_rel_w, rel_table, b_node, b_rel, attn3)

    acc, den = _sc_edge_pass(m_node, s_src.reshape(NPAD), s_rel.reshape(R),
                             m_rel, edge_index[0], edge_index[1], edge_rel)

    out_pad = _tc_combine(acc.reshape(NC, NPAD, D), den.reshape(NW, NPAD),
                          x_pad)
    return out_pad[:N]
